# baseline (speedup 1.0000x reference)
import jax
import jax.numpy as jnp
from jax.experimental import pallas as pl
from jax.experimental.pallas import tpu as pltpu

_N = 1024
_F = 128
_D = 128
_T = 2
_R = 2
_NT = 2
_E = 8
_BN = 256
_G = _N // _BN


def _gat_kernel(src_ref, dst_ref, adj_ref, w_ref, wlt_ref, wr_ref, b_ref,
                o_ref, hs_ref, el_ref):
    @pl.when(pl.program_id(1) == 0)
    def _prep():
        hs_ref[...] = jnp.dot(src_ref[...], w_ref[...],
                              preferred_element_type=jnp.float32).astype(jnp.bfloat16)
        el_ref[...] = jax.lax.dot_general(
            wlt_ref[...], src_ref[...], (((1,), (1,)), ((), ())),
            preferred_element_type=jnp.float32)

    er = jnp.dot(dst_ref[...], wr_ref[...],
                 preferred_element_type=jnp.float32)
    e = er + el_ref[...]
    e = jnp.where(e >= 0.0, e, 0.2 * e)
    mask = adj_ref[...] > 0.0
    e = jnp.where(mask, e, -1e30)
    emax = jnp.max(e, axis=-1, keepdims=True)
    p = jnp.where(mask, jnp.exp(e - emax), 0.0)
    denom = jnp.sum(p, axis=-1, keepdims=True)
    denom = jnp.where(denom > 0.0, denom, 1.0)
    attn = (p * pl.reciprocal(denom, approx=True)).astype(jnp.bfloat16)
    o_ref[...] = jnp.dot(attn, hs_ref[...],
                         preferred_element_type=jnp.float32) + b_ref[...]


def _fused_gat(src_all, dst_all, adj_all, w_all, wlt_all, wr_all, b_all):
    return pl.pallas_call(
        _gat_kernel,
        out_shape=jax.ShapeDtypeStruct((_E, _N, _D), jnp.float32),
        grid=(_E, _G),
        in_specs=[
            pl.BlockSpec((None, _N, _F), lambda e, i: (e, 0, 0)),
            pl.BlockSpec((None, _BN, _F), lambda e, i: (e, i, 0)),
            pl.BlockSpec((None, _BN, _N), lambda e, i: (e, i, 0)),
            pl.BlockSpec((None, _F, _D), lambda e, i: (e, 0, 0)),
            pl.BlockSpec((None, 1, _F), lambda e, i: (e, 0, 0)),
            pl.BlockSpec((None, _F, 1), lambda e, i: (e, 0, 0)),
            pl.BlockSpec((None, 1, _D), lambda e, i: (e, 0, 0)),
        ],
        out_specs=pl.BlockSpec((None, _BN, _D), lambda e, i: (e, i, 0)),
        scratch_shapes=[pltpu.VMEM((_N, _D), jnp.bfloat16),
                        pltpu.VMEM((1, _N), jnp.float32)],
        compiler_params=pltpu.CompilerParams(
            dimension_semantics=("parallel", "arbitrary")),
    )(src_all, dst_all, adj_all, w_all, wlt_all, wr_all, b_all)


def _pipe_kernel(h_ref, x_ref, w1_ref, b1_ref, w2_ref, pe_ref,
                 wp_ref, bp_ref, wqkv_ref, wf_ref, bf_ref,
                 wr_ref, br_ref, g_ref, be_ref, o_ref):
    xh = []
    for t in range(_T):
        amean = []
        for r in range(_R):
            z = jnp.tanh(jnp.dot(h_ref[t, r], w1_ref[t],
                                 preferred_element_type=jnp.float32) + b1_ref[t])
            amean.append(jnp.mean(z, axis=0, keepdims=True))
        means = [jax.lax.dot_general(amean[r], w2_ref[t],
                                     (((1,), (0,)), ((), ())),
                                     preferred_element_type=jnp.float32)[0, 0]
                 for r in range(_R)]
        m = jnp.maximum(means[0], means[1])
        e0 = jnp.exp(means[0] - m)
        e1 = jnp.exp(means[1] - m)
        inv = 1.0 / (e0 + e1)
        xh.append((e0 * inv) * h_ref[t, 0] + (e1 * inv) * h_ref[t, 1])

    q, k, v = [], [], []
    for t in range(_T):
        hf = jnp.dot(xh[t], wp_ref[...],
                     preferred_element_type=jnp.float32) + bp_ref[...] + pe_ref[t]
        qkv = jnp.dot(hf, wqkv_ref[...], preferred_element_type=jnp.float32)
        q.append(qkv[:, :_D])
        k.append(qkv[:, _D:2 * _D])
        v.append(qkv[:, 2 * _D:])
    for t in range(_T):
        s0 = jnp.sum(q[t] * k[0], axis=-1, keepdims=True)
        s1 = jnp.sum(q[t] * k[1], axis=-1, keepdims=True)
        m = jnp.maximum(s0, s1)
        p0 = jnp.exp(s0 - m)
        p1 = jnp.exp(s1 - m)
        inv = pl.reciprocal(p0 + p1, approx=True)
        hh = (p0 * inv) * v[0] + (p1 * inv) * v[1]
        ha = jnp.maximum(jnp.dot(hh, wf_ref[...],
                                 preferred_element_type=jnp.float32) + bf_ref[...], 0.0)
        res = jnp.dot(x_ref[:, t, :], wr_ref[...],
                      preferred_element_type=jnp.float32) + br_ref[...]
        y = ha + res
        mu = jnp.mean(y, axis=-1, keepdims=True)
        var = jnp.mean((y - mu) ** 2, axis=-1, keepdims=True)
        out = (y - mu) * jax.lax.rsqrt(var + 1e-5) * g_ref[...] + be_ref[...]
        o_ref[:, t * _D:(t + 1) * _D] = out


def _fused_pipe(h5, x_res, w1, b1, w2, pe, wp, bp, wqkv,
                wf_a, bf_a, wr_a, br_a, gamma, beta):
    shared3 = lambda i: (0, 0, 0)
    per3 = lambda i: (i, 0, 0)
    return pl.pallas_call(
        _pipe_kernel,
        out_shape=jax.ShapeDtypeStruct((_NT, _N, _T * _D), jnp.float32),
        grid=(_NT,),
        in_specs=[
            pl.BlockSpec((None, _T, _R, _N, _D), lambda i: (i, 0, 0, 0, 0)),
            pl.BlockSpec((None, _N, _T, _F), lambda i: (i, 0, 0, 0)),
            pl.BlockSpec((_T, _D, _D), shared3),
            pl.BlockSpec((_T, 1, _D), shared3),
            pl.BlockSpec((_T, _D, 1), shared3),
            pl.BlockSpec((None, _T, 1, _D), lambda i: (i, 0, 0, 0)),
            pl.BlockSpec((None, _D, _D), per3),
            pl.BlockSpec((None, 1, _D), per3),
            pl.BlockSpec((None, _D, 3 * _D), per3),
            pl.BlockSpec((None, _D, _D), per3),
            pl.BlockSpec((None, 1, _D), per3),
            pl.BlockSpec((None, _F, _D), per3),
            pl.BlockSpec((None, 1, _D), per3),
            pl.BlockSpec((None, 1, _D), per3),
            pl.BlockSpec((None, 1, _D), per3),
        ],
        out_specs=pl.BlockSpec((None, _N, _T * _D), per3),
        compiler_params=pltpu.CompilerParams(dimension_semantics=("parallel",)),
    )(h5, x_res, w1, b1, w2, pe, wp, bp, wqkv, wf_a, bf_a, wr_a, br_a, gamma, beta)


def kernel(nf__A__t0, nf__A__t1, nf__B__t0, nf__B__t1, adj__aa_t0, adj__ba_t0, adj__ab_t0, adj__bb_t0, adj__aa_t1, adj__ba_t1, adj__ab_t1, adj__bb_t1, intra__aa_t0__W, intra__aa_t0__attn_l, intra__aa_t0__attn_r, intra__aa_t0__bias, intra__ba_t0__W, intra__ba_t0__attn_l, intra__ba_t0__attn_r, intra__ba_t0__bias, intra__ab_t0__W, intra__ab_t0__attn_l, intra__ab_t0__attn_r, intra__ab_t0__bias, intra__bb_t0__W, intra__bb_t0__attn_l, intra__bb_t0__attn_r, intra__bb_t0__bias, intra__aa_t1__W, intra__aa_t1__attn_l, intra__aa_t1__attn_r, intra__aa_t1__bias, intra__ba_t1__W, intra__ba_t1__attn_l, intra__ba_t1__attn_r, intra__ba_t1__bias, intra__ab_t1__W, intra__ab_t1__attn_l, intra__ab_t1__attn_r, intra__ab_t1__bias, intra__bb_t1__W, intra__bb_t1__attn_l, intra__bb_t1__attn_r, intra__bb_t1__bias, inter__t0__W1, inter__t0__b1, inter__t0__w2, inter__t1__W1, inter__t1__b1, inter__t1__w2, cross__A__Wp, cross__A__bp, cross__A__Wq, cross__A__Wk, cross__A__Wv, cross__A__Wf, cross__A__bf, cross__A__pe, cross__B__Wp, cross__B__bp, cross__B__Wq, cross__B__Wk, cross__B__Wv, cross__B__Wf, cross__B__bf, cross__B__pe, res__A__W, res__A__b, res__A__res_weight, res__B__W, res__B__b, res__B__res_weight, norm__A__gamma, norm__A__beta, norm__B__gamma, norm__B__beta):
    bf16 = jnp.bfloat16
    srcs = [nf__A__t0, nf__B__t0, nf__A__t1, nf__B__t1,
            nf__A__t0, nf__B__t0, nf__A__t1, nf__B__t1]
    dsts = [nf__A__t0, nf__A__t0, nf__A__t1, nf__A__t1,
            nf__B__t0, nf__B__t0, nf__B__t1, nf__B__t1]
    adjs = [adj__aa_t0, adj__ba_t0, adj__aa_t1, adj__ba_t1,
            adj__ab_t0, adj__bb_t0, adj__ab_t1, adj__bb_t1]
    ws = [intra__aa_t0__W, intra__ba_t0__W, intra__aa_t1__W, intra__ba_t1__W,
          intra__ab_t0__W, intra__bb_t0__W, intra__ab_t1__W, intra__bb_t1__W]
    als = [intra__aa_t0__attn_l, intra__ba_t0__attn_l, intra__aa_t1__attn_l,
           intra__ba_t1__attn_l, intra__ab_t0__attn_l, intra__bb_t0__attn_l,
           intra__ab_t1__attn_l, intra__bb_t1__attn_l]
    ars = [intra__aa_t0__attn_r, intra__ba_t0__attn_r, intra__aa_t1__attn_r,
           intra__ba_t1__attn_r, intra__ab_t0__attn_r, intra__bb_t0__attn_r,
           intra__ab_t1__attn_r, intra__bb_t1__attn_r]
    bs = [intra__aa_t0__bias, intra__ba_t0__bias, intra__aa_t1__bias,
          intra__ba_t1__bias, intra__ab_t0__bias, intra__bb_t0__bias,
          intra__ab_t1__bias, intra__bb_t1__bias]

    src_all = jnp.stack(srcs).astype(bf16)
    dst_all = jnp.stack(dsts).astype(bf16)
    adj_all = jnp.stack(adjs).astype(bf16)
    w_all = jnp.stack(ws).astype(bf16)
    wlt_all = jnp.stack([al @ w.T for al, w in zip(als, ws)]).astype(bf16)
    wr_all = jnp.stack([w @ ar.T for ar, w in zip(ars, ws)]).astype(bf16)
    b_all = jnp.stack(bs)

    intra = _fused_gat(src_all, dst_all, adj_all, w_all, wlt_all, wr_all, b_all)
    h5 = intra.reshape(_NT, _T, _R, _N, _D)

    x_res = jnp.stack([jnp.stack([nf__A__t0, nf__A__t1], axis=1),
                       jnp.stack([nf__B__t0, nf__B__t1], axis=1)])
    w1 = jnp.stack([inter__t0__W1, inter__t1__W1])
    b1 = jnp.stack([inter__t0__b1, inter__t1__b1])
    w2 = jnp.stack([inter__t0__w2, inter__t1__w2])
    pe = jnp.stack([cross__A__pe, cross__B__pe])[:, :, None, :]
    wp = jnp.stack([cross__A__Wp, cross__B__Wp])
    bp = jnp.stack([cross__A__bp, cross__B__bp])
    wqkv = jnp.stack(
        [jnp.concatenate([cross__A__Wq, cross__A__Wk, cross__A__Wv], axis=1),
         jnp.concatenate([cross__B__Wq, cross__B__Wk, cross__B__Wv], axis=1)])
    alpha_a = jax.nn.sigmoid(res__A__res_weight)
    alpha_b = jax.nn.sigmoid(res__B__res_weight)
    wf_a = jnp.stack([cross__A__Wf * alpha_a, cross__B__Wf * alpha_b])
    bf_a = jnp.stack([cross__A__bf * alpha_a, cross__B__bf * alpha_b])
    wr_a = jnp.stack([res__A__W * (1.0 - alpha_a), res__B__W * (1.0 - alpha_b)])
    br_a = jnp.stack([res__A__b * (1.0 - alpha_a), res__B__b * (1.0 - alpha_b)])
    gamma = jnp.stack([norm__A__gamma, norm__B__gamma])
    beta = jnp.stack([norm__A__beta, norm__B__beta])

    out_all = _fused_pipe(h5, x_res, w1, b1, w2, pe, wp, bp, wqkv,
                          wf_a, bf_a, wr_a, br_a, gamma, beta)
    return {'A': {'t0': out_all[0, :, :_D], 't1': out_all[0, :, _D:]},
            'B': {'t0': out_all[1, :, :_D], 't1': out_all[1, :, _D:]}}


# speedup vs baseline: 1.4444x; 1.4444x over previous
import jax
import jax.numpy as jnp
from jax.experimental import pallas as pl
from jax.experimental.pallas import tpu as pltpu

_N = 1024
_F = 128
_D = 128
_T = 2
_R = 2
_NT = 2
_E = 8
_BN = 128
_G = _N // _BN

_DST_IDX = (0, 0, 1, 1, 2, 2, 3, 3)


def _prep_kernel(src_ref, w_ref, hs_ref):
    hs_ref[...] = jnp.dot(src_ref[...], w_ref[...],
                          preferred_element_type=jnp.float32).astype(jnp.bfloat16)


def _prep(src_all, w_all):
    return pl.pallas_call(
        _prep_kernel,
        out_shape=jax.ShapeDtypeStruct((_E, _N, _D), jnp.bfloat16),
        grid=(_E,),
        in_specs=[pl.BlockSpec((None, _N, _F), lambda e: (e, 0, 0)),
                  pl.BlockSpec((None, _F, _D), lambda e: (e, 0, 0))],
        out_specs=pl.BlockSpec((None, _N, _D), lambda e: (e, 0, 0)),
        compiler_params=pltpu.CompilerParams(dimension_semantics=("parallel",)),
    )(src_all, w_all)


def _gat_kernel(xa0_ref, xa1_ref, xb0_ref, xb1_ref,
                j0_ref, j1_ref, j2_ref, j3_ref, j4_ref, j5_ref, j6_ref, j7_ref,
                hs_ref, wr_ref, al_ref, b_ref, o_ref):
    dsts = (xa0_ref, xa1_ref, xb0_ref, xb1_ref)
    adjs = (j0_ref, j1_ref, j2_ref, j3_ref, j4_ref, j5_ref, j6_ref, j7_ref)
    for e in range(_E):
        hs_e = hs_ref[e]
        el = jax.lax.dot_general(al_ref[e], hs_e, (((1,), (1,)), ((), ())),
                                 preferred_element_type=jnp.float32)
        er = jnp.dot(dsts[_DST_IDX[e]][...], wr_ref[e],
                     preferred_element_type=jnp.float32)
        lg = er + el
        lg = jnp.maximum(lg, 0.2 * lg)
        p = jnp.exp(lg) * adjs[e][...]
        denom = jnp.sum(p, axis=-1, keepdims=True)
        sc = jnp.where(denom > 0.0, pl.reciprocal(denom, approx=True), 1.0)
        mm = jnp.dot(p.astype(jnp.bfloat16), hs_e,
                     preferred_element_type=jnp.float32)
        o_ref[e] = mm * sc + b_ref[e]


def _fused_gat(nf4, adj8, hs_all, wr_all, al_all, b_all):
    row_tile = lambda i: (i, 0)
    full3 = lambda i: (0, 0, 0)
    return pl.pallas_call(
        _gat_kernel,
        out_shape=jax.ShapeDtypeStruct((_E, _N, _D), jnp.float32),
        grid=(_G,),
        in_specs=(
            [pl.BlockSpec((_BN, _F), row_tile) for _ in range(4)] +
            [pl.BlockSpec((_BN, _N), row_tile) for _ in range(8)] +
            [pl.BlockSpec((_E, _N, _D), full3),
             pl.BlockSpec((_E, _F, 1), full3),
             pl.BlockSpec((_E, 1, _D), full3),
             pl.BlockSpec((_E, 1, _D), full3)]
        ),
        out_specs=pl.BlockSpec((_E, _BN, _D), lambda i: (0, i, 0)),
        compiler_params=pltpu.CompilerParams(dimension_semantics=("parallel",)),
    )(*nf4, *adj8, hs_all, wr_all, al_all, b_all)


def _pipe_kernel(h_ref, xa0_ref, xa1_ref, xb0_ref, xb1_ref,
                 w1_ref, b1_ref, w2_ref, pe_ref,
                 wp_ref, bp_ref, wqkv_ref, wf_ref, bf_ref,
                 wr_ref, br_ref, g_ref, be_ref, o_ref):
    is_a = pl.program_id(0) == 0
    xs = [jnp.where(is_a, xa0_ref[...], xb0_ref[...]),
          jnp.where(is_a, xa1_ref[...], xb1_ref[...])]
    xh = []
    for t in range(_T):
        amean = []
        for r in range(_R):
            z = jnp.tanh(jnp.dot(h_ref[2 * t + r], w1_ref[t],
                                 preferred_element_type=jnp.float32) + b1_ref[t])
            amean.append(jnp.mean(z, axis=0, keepdims=True))
        means = [jax.lax.dot_general(amean[r], w2_ref[t],
                                     (((1,), (0,)), ((), ())),
                                     preferred_element_type=jnp.float32)[0, 0]
                 for r in range(_R)]
        m = jnp.maximum(means[0], means[1])
        e0 = jnp.exp(means[0] - m)
        e1 = jnp.exp(means[1] - m)
        inv = 1.0 / (e0 + e1)
        xh.append((e0 * inv) * h_ref[2 * t] + (e1 * inv) * h_ref[2 * t + 1])

    q, k, v = [], [], []
    for t in range(_T):
        hf = jnp.dot(xh[t], wp_ref[...],
                     preferred_element_type=jnp.float32) + bp_ref[...] + pe_ref[t]
        qkv = jnp.dot(hf, wqkv_ref[...], preferred_element_type=jnp.float32)
        q.append(qkv[:, :_D])
        k.append(qkv[:, _D:2 * _D])
        v.append(qkv[:, 2 * _D:])
    for t in range(_T):
        s0 = jnp.sum(q[t] * k[0], axis=-1, keepdims=True)
        s1 = jnp.sum(q[t] * k[1], axis=-1, keepdims=True)
        m = jnp.maximum(s0, s1)
        p0 = jnp.exp(s0 - m)
        p1 = jnp.exp(s1 - m)
        inv = pl.reciprocal(p0 + p1, approx=True)
        hh = (p0 * inv) * v[0] + (p1 * inv) * v[1]
        ha = jnp.maximum(jnp.dot(hh, wf_ref[...],
                                 preferred_element_type=jnp.float32) + bf_ref[...], 0.0)
        res = jnp.dot(xs[t], wr_ref[...],
                      preferred_element_type=jnp.float32) + br_ref[...]
        y = ha + res
        mu = jnp.mean(y, axis=-1, keepdims=True)
        var = jnp.mean((y - mu) ** 2, axis=-1, keepdims=True)
        out = (y - mu) * jax.lax.rsqrt(var + 1e-5) * g_ref[...] + be_ref[...]
        o_ref[:, t * _D:(t + 1) * _D] = out


def _fused_pipe(h_all, nf4, w1, b1, w2, pe, wp, bp, wqkv,
                wf_a, bf_a, wr_a, br_a, gamma, beta):
    shared2 = lambda i: (0, 0)
    shared3 = lambda i: (0, 0, 0)
    per3 = lambda i: (i, 0, 0)
    return pl.pallas_call(
        _pipe_kernel,
        out_shape=jax.ShapeDtypeStruct((_NT, _N, _T * _D), jnp.float32),
        grid=(_NT,),
        in_specs=(
            [pl.BlockSpec((_T * _R, _N, _D), per3)] +
            [pl.BlockSpec((_N, _F), shared2) for _ in range(4)] +
            [pl.BlockSpec((_T, _D, _D), shared3),
             pl.BlockSpec((_T, 1, _D), shared3),
             pl.BlockSpec((_T, _D, 1), shared3),
             pl.BlockSpec((None, _T, 1, _D), lambda i: (i, 0, 0, 0)),
             pl.BlockSpec((None, _D, _D), per3),
             pl.BlockSpec((None, 1, _D), per3),
             pl.BlockSpec((None, _D, 3 * _D), per3),
             pl.BlockSpec((None, _D, _D), per3),
             pl.BlockSpec((None, 1, _D), per3),
             pl.BlockSpec((None, _F, _D), per3),
             pl.BlockSpec((None, 1, _D), per3),
             pl.BlockSpec((None, 1, _D), per3),
             pl.BlockSpec((None, 1, _D), per3)]
        ),
        out_specs=pl.BlockSpec((None, _N, _T * _D), per3),
        compiler_params=pltpu.CompilerParams(dimension_semantics=("parallel",)),
    )(h_all, *nf4, w1, b1, w2, pe, wp, bp, wqkv, wf_a, bf_a, wr_a, br_a, gamma, beta)


def kernel(nf__A__t0, nf__A__t1, nf__B__t0, nf__B__t1, adj__aa_t0, adj__ba_t0, adj__ab_t0, adj__bb_t0, adj__aa_t1, adj__ba_t1, adj__ab_t1, adj__bb_t1, intra__aa_t0__W, intra__aa_t0__attn_l, intra__aa_t0__attn_r, intra__aa_t0__bias, intra__ba_t0__W, intra__ba_t0__attn_l, intra__ba_t0__attn_r, intra__ba_t0__bias, intra__ab_t0__W, intra__ab_t0__attn_l, intra__ab_t0__attn_r, intra__ab_t0__bias, intra__bb_t0__W, intra__bb_t0__attn_l, intra__bb_t0__attn_r, intra__bb_t0__bias, intra__aa_t1__W, intra__aa_t1__attn_l, intra__aa_t1__attn_r, intra__aa_t1__bias, intra__ba_t1__W, intra__ba_t1__attn_l, intra__ba_t1__attn_r, intra__ba_t1__bias, intra__ab_t1__W, intra__ab_t1__attn_l, intra__ab_t1__attn_r, intra__ab_t1__bias, intra__bb_t1__W, intra__bb_t1__attn_l, intra__bb_t1__attn_r, intra__bb_t1__bias, inter__t0__W1, inter__t0__b1, inter__t0__w2, inter__t1__W1, inter__t1__b1, inter__t1__w2, cross__A__Wp, cross__A__bp, cross__A__Wq, cross__A__Wk, cross__A__Wv, cross__A__Wf, cross__A__bf, cross__A__pe, cross__B__Wp, cross__B__bp, cross__B__Wq, cross__B__Wk, cross__B__Wv, cross__B__Wf, cross__B__bf, cross__B__pe, res__A__W, res__A__b, res__A__res_weight, res__B__W, res__B__b, res__B__res_weight, norm__A__gamma, norm__A__beta, norm__B__gamma, norm__B__beta):
    bf16 = jnp.bfloat16
    srcs = [nf__A__t0, nf__B__t0, nf__A__t1, nf__B__t1,
            nf__A__t0, nf__B__t0, nf__A__t1, nf__B__t1]
    adjs = [adj__aa_t0, adj__ba_t0, adj__aa_t1, adj__ba_t1,
            adj__ab_t0, adj__bb_t0, adj__ab_t1, adj__bb_t1]
    ws = [intra__aa_t0__W, intra__ba_t0__W, intra__aa_t1__W, intra__ba_t1__W,
          intra__ab_t0__W, intra__bb_t0__W, intra__ab_t1__W, intra__bb_t1__W]
    als = [intra__aa_t0__attn_l, intra__ba_t0__attn_l, intra__aa_t1__attn_l,
           intra__ba_t1__attn_l, intra__ab_t0__attn_l, intra__bb_t0__attn_l,
           intra__ab_t1__attn_l, intra__bb_t1__attn_l]
    ars = [intra__aa_t0__attn_r, intra__ba_t0__attn_r, intra__aa_t1__attn_r,
           intra__ba_t1__attn_r, intra__ab_t0__attn_r, intra__bb_t0__attn_r,
           intra__ab_t1__attn_r, intra__bb_t1__attn_r]
    bs = [intra__aa_t0__bias, intra__ba_t0__bias, intra__aa_t1__bias,
          intra__ba_t1__bias, intra__ab_t0__bias, intra__bb_t0__bias,
          intra__ab_t1__bias, intra__bb_t1__bias]

    src_all = jnp.stack(srcs).astype(bf16)
    w_all = jnp.stack(ws).astype(bf16)
    hs_all = _prep(src_all, w_all)

    wr_all = jnp.stack([w @ ar.T for ar, w in zip(ars, ws)])
    al_all = jnp.stack(als).astype(bf16)
    b_all = jnp.stack(bs)

    nf4 = [nf__A__t0, nf__A__t1, nf__B__t0, nf__B__t1]
    intra = _fused_gat(nf4, adjs, hs_all, wr_all, al_all, b_all)

    w1 = jnp.stack([inter__t0__W1, inter__t1__W1])
    b1 = jnp.stack([inter__t0__b1, inter__t1__b1])
    w2 = jnp.stack([inter__t0__w2, inter__t1__w2])
    pe = jnp.stack([cross__A__pe, cross__B__pe])[:, :, None, :]
    wp = jnp.stack([cross__A__Wp, cross__B__Wp])
    bp = jnp.stack([cross__A__bp, cross__B__bp])
    wqkv = jnp.stack(
        [jnp.concatenate([cross__A__Wq, cross__A__Wk, cross__A__Wv], axis=1),
         jnp.concatenate([cross__B__Wq, cross__B__Wk, cross__B__Wv], axis=1)])
    alpha_a = jax.nn.sigmoid(res__A__res_weight)
    alpha_b = jax.nn.sigmoid(res__B__res_weight)
    wf_a = jnp.stack([cross__A__Wf * alpha_a, cross__B__Wf * alpha_b])
    bf_a = jnp.stack([cross__A__bf * alpha_a, cross__B__bf * alpha_b])
    wr_a = jnp.stack([res__A__W * (1.0 - alpha_a), res__B__W * (1.0 - alpha_b)])
    br_a = jnp.stack([res__A__b * (1.0 - alpha_a), res__B__b * (1.0 - alpha_b)])
    gamma = jnp.stack([norm__A__gamma, norm__B__gamma])
    beta = jnp.stack([norm__A__beta, norm__B__beta])

    out_all = _fused_pipe(intra, nf4, w1, b1, w2, pe, wp, bp, wqkv,
                          wf_a, bf_a, wr_a, br_a, gamma, beta)
    return {'A': {'t0': out_all[0, :, :_D], 't1': out_all[0, :, _D:]},
            'B': {'t0': out_all[1, :, :_D], 't1': out_all[1, :, _D:]}}


# speedup vs baseline: 1.6228x; 1.1235x over previous
import jax
import jax.numpy as jnp
from jax.experimental import pallas as pl
from jax.experimental.pallas import tpu as pltpu

_N = 1024
_F = 128
_D = 128
_T = 2
_R = 2
_NT = 2
_E = 8
_BN = 256
_G = _N // _BN

_DST_IDX = (0, 0, 1, 1, 2, 2, 3, 3)


def _prep_kernel(src_ref, w_ref, hs_ref):
    hs_ref[...] = jnp.dot(src_ref[...], w_ref[...],
                          preferred_element_type=jnp.float32).astype(jnp.bfloat16)


def _prep(src_all, w_all):
    return pl.pallas_call(
        _prep_kernel,
        out_shape=jax.ShapeDtypeStruct((_E, _N, _D), jnp.bfloat16),
        grid=(_E,),
        in_specs=[pl.BlockSpec((None, _N, _F), lambda e: (e, 0, 0)),
                  pl.BlockSpec((None, _F, _D), lambda e: (e, 0, 0))],
        out_specs=pl.BlockSpec((None, _N, _D), lambda e: (e, 0, 0)),
        compiler_params=pltpu.CompilerParams(dimension_semantics=("parallel",)),
    )(src_all, w_all)


def _gat_kernel(xa0_ref, xa1_ref, xb0_ref, xb1_ref,
                j0_ref, j1_ref, j2_ref, j3_ref, j4_ref, j5_ref, j6_ref, j7_ref,
                hs_ref, wr_ref, al_ref, b_ref, o_ref):
    dsts = (xa0_ref, xa1_ref, xb0_ref, xb1_ref)
    adjs = (j0_ref, j1_ref, j2_ref, j3_ref, j4_ref, j5_ref, j6_ref, j7_ref)
    for e in range(_E):
        hs_e = hs_ref[e]
        el = jax.lax.dot_general(al_ref[e], hs_e, (((1,), (1,)), ((), ())),
                                 preferred_element_type=jnp.float32)
        er = jnp.dot(dsts[_DST_IDX[e]][...], wr_ref[e],
                     preferred_element_type=jnp.float32)
        lg = er + el
        lg = jnp.maximum(lg, 0.2 * lg)
        p = jnp.exp(lg) * adjs[e][...]
        denom = jnp.sum(p, axis=-1, keepdims=True)
        sc = jnp.where(denom > 0.0, pl.reciprocal(denom, approx=True), 1.0)
        mm = jnp.dot(p.astype(jnp.bfloat16), hs_e,
                     preferred_element_type=jnp.float32)
        o_ref[e] = mm * sc + b_ref[e]


def _fused_gat(nf4, adj8, hs_all, wr_all, al_all, b_all):
    row_tile = lambda i: (i, 0)
    full3 = lambda i: (0, 0, 0)
    return pl.pallas_call(
        _gat_kernel,
        out_shape=jax.ShapeDtypeStruct((_E, _N, _D), jnp.float32),
        grid=(_G,),
        in_specs=(
            [pl.BlockSpec((_BN, _F), row_tile) for _ in range(4)] +
            [pl.BlockSpec((_BN, _N), row_tile) for _ in range(8)] +
            [pl.BlockSpec((_E, _N, _D), full3),
             pl.BlockSpec((_E, _F, 1), full3),
             pl.BlockSpec((_E, 1, _D), full3),
             pl.BlockSpec((_E, 1, _D), full3)]
        ),
        out_specs=pl.BlockSpec((_E, _BN, _D), lambda i: (0, i, 0)),
        compiler_params=pltpu.CompilerParams(dimension_semantics=("parallel",)),
    )(*nf4, *adj8, hs_all, wr_all, al_all, b_all)


def _pipe_kernel(h_ref, xa0_ref, xa1_ref, xb0_ref, xb1_ref,
                 w1_ref, b1_ref, w2_ref, pe_ref,
                 wp_ref, bp_ref, wqkv_ref, wf_ref, bf_ref,
                 wr_ref, br_ref, g_ref, be_ref, o_ref):
    is_a = pl.program_id(0) == 0
    xs = [jnp.where(is_a, xa0_ref[...], xb0_ref[...]),
          jnp.where(is_a, xa1_ref[...], xb1_ref[...])]
    xh = []
    for t in range(_T):
        amean = []
        for r in range(_R):
            z = jnp.tanh(jnp.dot(h_ref[2 * t + r], w1_ref[t],
                                 preferred_element_type=jnp.float32) + b1_ref[t])
            amean.append(jnp.mean(z, axis=0, keepdims=True))
        means = [jax.lax.dot_general(amean[r], w2_ref[t],
                                     (((1,), (0,)), ((), ())),
                                     preferred_element_type=jnp.float32)[0, 0]
                 for r in range(_R)]
        m = jnp.maximum(means[0], means[1])
        e0 = jnp.exp(means[0] - m)
        e1 = jnp.exp(means[1] - m)
        inv = 1.0 / (e0 + e1)
        xh.append((e0 * inv) * h_ref[2 * t] + (e1 * inv) * h_ref[2 * t + 1])

    q, k, v = [], [], []
    for t in range(_T):
        hf = jnp.dot(xh[t], wp_ref[...],
                     preferred_element_type=jnp.float32) + bp_ref[...] + pe_ref[t]
        qkv = jnp.dot(hf, wqkv_ref[...], preferred_element_type=jnp.float32)
        q.append(qkv[:, :_D])
        k.append(qkv[:, _D:2 * _D])
        v.append(qkv[:, 2 * _D:])
    for t in range(_T):
        s0 = jnp.sum(q[t] * k[0], axis=-1, keepdims=True)
        s1 = jnp.sum(q[t] * k[1], axis=-1, keepdims=True)
        m = jnp.maximum(s0, s1)
        p0 = jnp.exp(s0 - m)
        p1 = jnp.exp(s1 - m)
        inv = pl.reciprocal(p0 + p1, approx=True)
        hh = (p0 * inv) * v[0] + (p1 * inv) * v[1]
        ha = jnp.maximum(jnp.dot(hh, wf_ref[...],
                                 preferred_element_type=jnp.float32) + bf_ref[...], 0.0)
        res = jnp.dot(xs[t], wr_ref[...],
                      preferred_element_type=jnp.float32) + br_ref[...]
        y = ha + res
        mu = jnp.mean(y, axis=-1, keepdims=True)
        var = jnp.mean((y - mu) ** 2, axis=-1, keepdims=True)
        out = (y - mu) * jax.lax.rsqrt(var + 1e-5) * g_ref[...] + be_ref[...]
        o_ref[:, t * _D:(t + 1) * _D] = out


def _fused_pipe(h_all, nf4, w1, b1, w2, pe, wp, bp, wqkv,
                wf_a, bf_a, wr_a, br_a, gamma, beta):
    shared2 = lambda i: (0, 0)
    shared3 = lambda i: (0, 0, 0)
    per3 = lambda i: (i, 0, 0)
    return pl.pallas_call(
        _pipe_kernel,
        out_shape=jax.ShapeDtypeStruct((_NT, _N, _T * _D), jnp.float32),
        grid=(_NT,),
        in_specs=(
            [pl.BlockSpec((_T * _R, _N, _D), per3)] +
            [pl.BlockSpec((_N, _F), shared2) for _ in range(4)] +
            [pl.BlockSpec((_T, _D, _D), shared3),
             pl.BlockSpec((_T, 1, _D), shared3),
             pl.BlockSpec((_T, _D, 1), shared3),
             pl.BlockSpec((None, _T, 1, _D), lambda i: (i, 0, 0, 0)),
             pl.BlockSpec((None, _D, _D), per3),
             pl.BlockSpec((None, 1, _D), per3),
             pl.BlockSpec((None, _D, 3 * _D), per3),
             pl.BlockSpec((None, _D, _D), per3),
             pl.BlockSpec((None, 1, _D), per3),
             pl.BlockSpec((None, _F, _D), per3),
             pl.BlockSpec((None, 1, _D), per3),
             pl.BlockSpec((None, 1, _D), per3),
             pl.BlockSpec((None, 1, _D), per3)]
        ),
        out_specs=pl.BlockSpec((None, _N, _T * _D), per3),
        compiler_params=pltpu.CompilerParams(dimension_semantics=("parallel",)),
    )(h_all, *nf4, w1, b1, w2, pe, wp, bp, wqkv, wf_a, bf_a, wr_a, br_a, gamma, beta)


def kernel(nf__A__t0, nf__A__t1, nf__B__t0, nf__B__t1, adj__aa_t0, adj__ba_t0, adj__ab_t0, adj__bb_t0, adj__aa_t1, adj__ba_t1, adj__ab_t1, adj__bb_t1, intra__aa_t0__W, intra__aa_t0__attn_l, intra__aa_t0__attn_r, intra__aa_t0__bias, intra__ba_t0__W, intra__ba_t0__attn_l, intra__ba_t0__attn_r, intra__ba_t0__bias, intra__ab_t0__W, intra__ab_t0__attn_l, intra__ab_t0__attn_r, intra__ab_t0__bias, intra__bb_t0__W, intra__bb_t0__attn_l, intra__bb_t0__attn_r, intra__bb_t0__bias, intra__aa_t1__W, intra__aa_t1__attn_l, intra__aa_t1__attn_r, intra__aa_t1__bias, intra__ba_t1__W, intra__ba_t1__attn_l, intra__ba_t1__attn_r, intra__ba_t1__bias, intra__ab_t1__W, intra__ab_t1__attn_l, intra__ab_t1__attn_r, intra__ab_t1__bias, intra__bb_t1__W, intra__bb_t1__attn_l, intra__bb_t1__attn_r, intra__bb_t1__bias, inter__t0__W1, inter__t0__b1, inter__t0__w2, inter__t1__W1, inter__t1__b1, inter__t1__w2, cross__A__Wp, cross__A__bp, cross__A__Wq, cross__A__Wk, cross__A__Wv, cross__A__Wf, cross__A__bf, cross__A__pe, cross__B__Wp, cross__B__bp, cross__B__Wq, cross__B__Wk, cross__B__Wv, cross__B__Wf, cross__B__bf, cross__B__pe, res__A__W, res__A__b, res__A__res_weight, res__B__W, res__B__b, res__B__res_weight, norm__A__gamma, norm__A__beta, norm__B__gamma, norm__B__beta):
    bf16 = jnp.bfloat16
    srcs = [nf__A__t0, nf__B__t0, nf__A__t1, nf__B__t1,
            nf__A__t0, nf__B__t0, nf__A__t1, nf__B__t1]
    adjs = [adj__aa_t0, adj__ba_t0, adj__aa_t1, adj__ba_t1,
            adj__ab_t0, adj__bb_t0, adj__ab_t1, adj__bb_t1]
    ws = [intra__aa_t0__W, intra__ba_t0__W, intra__aa_t1__W, intra__ba_t1__W,
          intra__ab_t0__W, intra__bb_t0__W, intra__ab_t1__W, intra__bb_t1__W]
    als = [intra__aa_t0__attn_l, intra__ba_t0__attn_l, intra__aa_t1__attn_l,
           intra__ba_t1__attn_l, intra__ab_t0__attn_l, intra__bb_t0__attn_l,
           intra__ab_t1__attn_l, intra__bb_t1__attn_l]
    ars = [intra__aa_t0__attn_r, intra__ba_t0__attn_r, intra__aa_t1__attn_r,
           intra__ba_t1__attn_r, intra__ab_t0__attn_r, intra__bb_t0__attn_r,
           intra__ab_t1__attn_r, intra__bb_t1__attn_r]
    bs = [intra__aa_t0__bias, intra__ba_t0__bias, intra__aa_t1__bias,
          intra__ba_t1__bias, intra__ab_t0__bias, intra__bb_t0__bias,
          intra__ab_t1__bias, intra__bb_t1__bias]

    src_all = jnp.stack(srcs).astype(bf16)
    w_all = jnp.stack(ws).astype(bf16)
    hs_all = _prep(src_all, w_all)

    wr_all = jnp.stack([w @ ar.T for ar, w in zip(ars, ws)])
    al_all = jnp.stack(als).astype(bf16)
    b_all = jnp.stack(bs)

    nf4 = [nf__A__t0, nf__A__t1, nf__B__t0, nf__B__t1]
    intra = _fused_gat(nf4, adjs, hs_all, wr_all, al_all, b_all)

    w1 = jnp.stack([inter__t0__W1, inter__t1__W1])
    b1 = jnp.stack([inter__t0__b1, inter__t1__b1])
    w2 = jnp.stack([inter__t0__w2, inter__t1__w2])
    pe = jnp.stack([cross__A__pe, cross__B__pe])[:, :, None, :]
    wp = jnp.stack([cross__A__Wp, cross__B__Wp])
    bp = jnp.stack([cross__A__bp, cross__B__bp])
    wqkv = jnp.stack(
        [jnp.concatenate([cross__A__Wq, cross__A__Wk, cross__A__Wv], axis=1),
         jnp.concatenate([cross__B__Wq, cross__B__Wk, cross__B__Wv], axis=1)])
    alpha_a = jax.nn.sigmoid(res__A__res_weight)
    alpha_b = jax.nn.sigmoid(res__B__res_weight)
    wf_a = jnp.stack([cross__A__Wf * alpha_a, cross__B__Wf * alpha_b])
    bf_a = jnp.stack([cross__A__bf * alpha_a, cross__B__bf * alpha_b])
    wr_a = jnp.stack([res__A__W * (1.0 - alpha_a), res__B__W * (1.0 - alpha_b)])
    br_a = jnp.stack([res__A__b * (1.0 - alpha_a), res__B__b * (1.0 - alpha_b)])
    gamma = jnp.stack([norm__A__gamma, norm__B__gamma])
    beta = jnp.stack([norm__A__beta, norm__B__beta])

    out_all = _fused_pipe(intra, nf4, w1, b1, w2, pe, wp, bp, wqkv,
                          wf_a, bf_a, wr_a, br_a, gamma, beta)
    return {'A': {'t0': out_all[0, :, :_D], 't1': out_all[0, :, _D:]},
            'B': {'t0': out_all[1, :, :_D], 't1': out_all[1, :, _D:]}}


# speedup vs baseline: 2.2676x; 1.3973x over previous
import jax
import jax.numpy as jnp
from jax.experimental import pallas as pl
from jax.experimental.pallas import tpu as pltpu

_N = 1024
_F = 128
_D = 128
_T = 2
_R = 2
_NT = 2
_E = 8
_BN = 256
_G = _N // _BN

_SRC_IDX = (0, 2, 1, 3, 0, 2, 1, 3)
_DST_IDX = (0, 0, 1, 1, 2, 2, 3, 3)


def _gat_kernel(*refs):
    nf_refs = refs[0:4]
    adj_refs = refs[4:12]
    w_refs = refs[12:20]
    al_refs = refs[20:28]
    ar_refs = refs[28:36]
    b_refs = refs[36:44]
    o_ref = refs[44]
    i = pl.program_id(0)
    nf_bf = [x[...].astype(jnp.bfloat16) for x in nf_refs]
    for e in range(_E):
        w_bf = w_refs[e][...].astype(jnp.bfloat16)
        hs32 = jnp.dot(nf_bf[_SRC_IDX[e]], w_bf,
                       preferred_element_type=jnp.float32)
        hs_e = hs32.astype(jnp.bfloat16)
        el = jax.lax.dot_general(al_refs[e][...], hs32,
                                 (((1,), (1,)), ((), ())),
                                 preferred_element_type=jnp.float32)
        wr_col = jax.lax.dot_general(w_refs[e][...], ar_refs[e][...],
                                     (((1,), (1,)), ((), ())),
                                     preferred_element_type=jnp.float32)
        dst32 = nf_refs[_DST_IDX[e]][pl.ds(i * _BN, _BN), :]
        er = jnp.dot(dst32, wr_col,
                     preferred_element_type=jnp.float32)
        lg = er + el
        lg = jnp.maximum(lg, 0.2 * lg)
        p = jnp.exp(lg) * adj_refs[e][...]
        denom = jnp.sum(p, axis=-1, keepdims=True)
        sc = jnp.where(denom > 0.0, pl.reciprocal(denom, approx=True), 1.0)
        mm = jnp.dot(p.astype(jnp.bfloat16), hs_e,
                     preferred_element_type=jnp.float32)
        o_ref[e] = mm * sc + b_refs[e][...]


def _fused_gat(nf4, adj8, w8, al8, ar8, b8):
    full2 = lambda i: (0, 0)
    row_tile = lambda i: (i, 0)
    return pl.pallas_call(
        _gat_kernel,
        out_shape=jax.ShapeDtypeStruct((_E, _N, _D), jnp.float32),
        grid=(_G,),
        in_specs=(
            [pl.BlockSpec((_N, _F), full2) for _ in range(4)] +
            [pl.BlockSpec((_BN, _N), row_tile) for _ in range(8)] +
            [pl.BlockSpec((_F, _D), full2) for _ in range(8)] +
            [pl.BlockSpec((1, _D), full2) for _ in range(8)] +
            [pl.BlockSpec((1, _D), full2) for _ in range(8)] +
            [pl.BlockSpec((1, _D), full2) for _ in range(8)]
        ),
        out_specs=pl.BlockSpec((_E, _BN, _D), lambda i: (0, i, 0)),
        compiler_params=pltpu.CompilerParams(dimension_semantics=("parallel",)),
    )(*nf4, *adj8, *w8, *al8, *ar8, *b8)


def _pipe_kernel(*refs):
    h_ref = refs[0]
    nf_refs = refs[1:5]
    w1_refs, b1_refs, w2_refs = refs[5:7], refs[7:9], refs[9:11]
    (wpa, wpb, bpa, bpb, wqa, wqb, wka, wkb, wva, wvb, wfa, wfb,
     bfa, bfb, pea, peb, wra, wrb, bra, brb, rwa, rwb,
     ga, gb, bea, beb) = refs[11:37]
    o_ref = refs[37]

    is_a = pl.program_id(0) == 0
    sel = lambda a, b: jnp.where(is_a, a[...], b[...])
    xs = [jnp.where(is_a, nf_refs[0][...], nf_refs[2][...]),
          jnp.where(is_a, nf_refs[1][...], nf_refs[3][...])]
    alpha = jax.nn.sigmoid(jnp.where(is_a, rwa[0, 0], rwb[0, 0]))
    wp, bp = sel(wpa, wpb), sel(bpa, bpb)
    wqkv = jnp.concatenate([sel(wqa, wqb), sel(wka, wkb), sel(wva, wvb)], axis=1)
    wf, bf = sel(wfa, wfb) * alpha, sel(bfa, bfb) * alpha
    pe = sel(pea, peb)
    wr, br = sel(wra, wrb) * (1.0 - alpha), sel(bra, brb) * (1.0 - alpha)
    g, be = sel(ga, gb), sel(bea, beb)

    xh = []
    for t in range(_T):
        amean = []
        for r in range(_R):
            z = jnp.tanh(jnp.dot(h_ref[2 * t + r], w1_refs[t][...],
                                 preferred_element_type=jnp.float32) + b1_refs[t][...])
            amean.append(jnp.mean(z, axis=0, keepdims=True))
        means = [jax.lax.dot_general(amean[r], w2_refs[t][...],
                                     (((1,), (0,)), ((), ())),
                                     preferred_element_type=jnp.float32)[0, 0]
                 for r in range(_R)]
        m = jnp.maximum(means[0], means[1])
        e0 = jnp.exp(means[0] - m)
        e1 = jnp.exp(means[1] - m)
        inv = 1.0 / (e0 + e1)
        xh.append((e0 * inv) * h_ref[2 * t] + (e1 * inv) * h_ref[2 * t + 1])

    q, k, v = [], [], []
    for t in range(_T):
        hf = jnp.dot(xh[t], wp,
                     preferred_element_type=jnp.float32) + bp + pe[t:t + 1, :]
        qkv = jnp.dot(hf, wqkv, preferred_element_type=jnp.float32)
        q.append(qkv[:, :_D])
        k.append(qkv[:, _D:2 * _D])
        v.append(qkv[:, 2 * _D:])
    for t in range(_T):
        s0 = jnp.sum(q[t] * k[0], axis=-1, keepdims=True)
        s1 = jnp.sum(q[t] * k[1], axis=-1, keepdims=True)
        m = jnp.maximum(s0, s1)
        p0 = jnp.exp(s0 - m)
        p1 = jnp.exp(s1 - m)
        inv = pl.reciprocal(p0 + p1, approx=True)
        hh = (p0 * inv) * v[0] + (p1 * inv) * v[1]
        ha = jnp.maximum(jnp.dot(hh, wf,
                                 preferred_element_type=jnp.float32) + bf, 0.0)
        res = jnp.dot(xs[t], wr, preferred_element_type=jnp.float32) + br
        y = ha + res
        mu = jnp.mean(y, axis=-1, keepdims=True)
        var = jnp.mean((y - mu) ** 2, axis=-1, keepdims=True)
        out = (y - mu) * jax.lax.rsqrt(var + 1e-5) * g + be
        o_ref[:, t * _D:(t + 1) * _D] = out


def _fused_pipe(h_all, nf4, w1s, b1s, w2s, wnt):
    shared2 = lambda i: (0, 0)
    per3 = lambda i: (i, 0, 0)
    return pl.pallas_call(
        _pipe_kernel,
        out_shape=jax.ShapeDtypeStruct((_NT, _N, _T * _D), jnp.float32),
        grid=(_NT,),
        in_specs=(
            [pl.BlockSpec((_T * _R, _N, _D), per3)] +
            [pl.BlockSpec((_N, _F), shared2) for _ in range(4)] +
            [pl.BlockSpec((_D, _D), shared2) for _ in range(2)] +
            [pl.BlockSpec((1, _D), shared2) for _ in range(2)] +
            [pl.BlockSpec((_D, 1), shared2) for _ in range(2)] +
            [pl.BlockSpec((_D, _D), shared2) for _ in range(2)] +
            [pl.BlockSpec((1, _D), shared2) for _ in range(2)] +
            [pl.BlockSpec((_D, _D), shared2) for _ in range(6)] +
            [pl.BlockSpec((_D, _D), shared2) for _ in range(2)] +
            [pl.BlockSpec((1, _D), shared2) for _ in range(2)] +
            [pl.BlockSpec((_T, _D), shared2) for _ in range(2)] +
            [pl.BlockSpec((_F, _D), shared2) for _ in range(2)] +
            [pl.BlockSpec((1, _D), shared2) for _ in range(2)] +
            [pl.BlockSpec((1, 1), shared2) for _ in range(2)] +
            [pl.BlockSpec((1, _D), shared2) for _ in range(2)] +
            [pl.BlockSpec((1, _D), shared2) for _ in range(2)]
        ),
        out_specs=pl.BlockSpec((None, _N, _T * _D), per3),
        compiler_params=pltpu.CompilerParams(dimension_semantics=("parallel",)),
    )(h_all, *nf4, *w1s, *b1s, *w2s, *wnt)


def kernel(nf__A__t0, nf__A__t1, nf__B__t0, nf__B__t1, adj__aa_t0, adj__ba_t0, adj__ab_t0, adj__bb_t0, adj__aa_t1, adj__ba_t1, adj__ab_t1, adj__bb_t1, intra__aa_t0__W, intra__aa_t0__attn_l, intra__aa_t0__attn_r, intra__aa_t0__bias, intra__ba_t0__W, intra__ba_t0__attn_l, intra__ba_t0__attn_r, intra__ba_t0__bias, intra__ab_t0__W, intra__ab_t0__attn_l, intra__ab_t0__attn_r, intra__ab_t0__bias, intra__bb_t0__W, intra__bb_t0__attn_l, intra__bb_t0__attn_r, intra__bb_t0__bias, intra__aa_t1__W, intra__aa_t1__attn_l, intra__aa_t1__attn_r, intra__aa_t1__bias, intra__ba_t1__W, intra__ba_t1__attn_l, intra__ba_t1__attn_r, intra__ba_t1__bias, intra__ab_t1__W, intra__ab_t1__attn_l, intra__ab_t1__attn_r, intra__ab_t1__bias, intra__bb_t1__W, intra__bb_t1__attn_l, intra__bb_t1__attn_r, intra__bb_t1__bias, inter__t0__W1, inter__t0__b1, inter__t0__w2, inter__t1__W1, inter__t1__b1, inter__t1__w2, cross__A__Wp, cross__A__bp, cross__A__Wq, cross__A__Wk, cross__A__Wv, cross__A__Wf, cross__A__bf, cross__A__pe, cross__B__Wp, cross__B__bp, cross__B__Wq, cross__B__Wk, cross__B__Wv, cross__B__Wf, cross__B__bf, cross__B__pe, res__A__W, res__A__b, res__A__res_weight, res__B__W, res__B__b, res__B__res_weight, norm__A__gamma, norm__A__beta, norm__B__gamma, norm__B__beta):
    nf4 = [nf__A__t0, nf__A__t1, nf__B__t0, nf__B__t1]
    adj8 = [adj__aa_t0, adj__ba_t0, adj__aa_t1, adj__ba_t1,
            adj__ab_t0, adj__bb_t0, adj__ab_t1, adj__bb_t1]
    w8 = [intra__aa_t0__W, intra__ba_t0__W, intra__aa_t1__W, intra__ba_t1__W,
          intra__ab_t0__W, intra__bb_t0__W, intra__ab_t1__W, intra__bb_t1__W]
    al8 = [intra__aa_t0__attn_l, intra__ba_t0__attn_l, intra__aa_t1__attn_l,
           intra__ba_t1__attn_l, intra__ab_t0__attn_l, intra__bb_t0__attn_l,
           intra__ab_t1__attn_l, intra__bb_t1__attn_l]
    ar8 = [intra__aa_t0__attn_r, intra__ba_t0__attn_r, intra__aa_t1__attn_r,
           intra__ba_t1__attn_r, intra__ab_t0__attn_r, intra__bb_t0__attn_r,
           intra__ab_t1__attn_r, intra__bb_t1__attn_r]
    b8 = [intra__aa_t0__bias, intra__ba_t0__bias, intra__aa_t1__bias,
          intra__ba_t1__bias, intra__ab_t0__bias, intra__bb_t0__bias,
          intra__ab_t1__bias, intra__bb_t1__bias]

    intra = _fused_gat(nf4, adj8, w8, al8, ar8, b8)

    wnt = [cross__A__Wp, cross__B__Wp, cross__A__bp, cross__B__bp,
           cross__A__Wq, cross__B__Wq, cross__A__Wk, cross__B__Wk,
           cross__A__Wv, cross__B__Wv, cross__A__Wf, cross__B__Wf,
           cross__A__bf, cross__B__bf, cross__A__pe, cross__B__pe,
           res__A__W, res__B__W, res__A__b, res__B__b,
           res__A__res_weight, res__B__res_weight,
           norm__A__gamma, norm__B__gamma, norm__A__beta, norm__B__beta]
    out_all = _fused_pipe(intra, nf4,
                          [inter__t0__W1, inter__t1__W1],
                          [inter__t0__b1, inter__t1__b1],
                          [inter__t0__w2, inter__t1__w2], wnt)
    return {'A': {'t0': out_all[0, :, :_D], 't1': out_all[0, :, _D:]},
            'B': {'t0': out_all[1, :, :_D], 't1': out_all[1, :, _D:]}}


# speedup vs baseline: 2.2889x; 1.0094x over previous
import jax
import jax.numpy as jnp
from jax.experimental import pallas as pl
from jax.experimental.pallas import tpu as pltpu

_N = 1024
_F = 128
_D = 128
_T = 2
_R = 2
_NT = 2
_E = 8
_BN = 256
_G = _N // _BN

_SRC_IDX = (0, 2, 1, 3, 0, 2, 1, 3)


def _htgnn_kernel(*refs):
    nf_refs = refs[0:4]
    adj_refs = refs[4:12]
    w_refs = refs[12:20]
    al_refs = refs[20:28]
    ar_refs = refs[28:36]
    b_refs = refs[36:44]
    w1_refs, b1_refs, w2_refs = refs[44:46], refs[46:48], refs[48:50]
    (wpa, wpb, bpa, bpb, wqa, wqb, wka, wkb, wva, wvb, wfa, wfb,
     bfa, bfb, pea, peb, wra, wrb, bra, brb, rwa, rwb,
     ga, gb, bea, beb) = refs[50:76]
    o_ref = refs[76]
    h_scr = refs[77]

    nt = pl.program_id(0)
    i = pl.program_id(1)
    is_a = nt == 0
    sel = lambda a, b: jnp.where(is_a, a[...], b[...])

    nf_bf = [x[...].astype(jnp.bfloat16) for x in nf_refs]
    xs = [jnp.where(is_a, nf_refs[0][...], nf_refs[2][...]),
          jnp.where(is_a, nf_refs[1][...], nf_refs[3][...])]

    for el_idx in range(4):
        ea, eb = el_idx, el_idx + 4
        w32 = sel(w_refs[ea], w_refs[eb])
        w_bf = w32.astype(jnp.bfloat16)
        src_bf = jnp.where(is_a, nf_bf[_SRC_IDX[ea]], nf_bf[_SRC_IDX[eb]])
        hs32 = jnp.dot(src_bf, w_bf, preferred_element_type=jnp.float32)
        hs_e = hs32.astype(jnp.bfloat16)
        al = sel(al_refs[ea], al_refs[eb])
        el = jax.lax.dot_general(al, hs32, (((1,), (1,)), ((), ())),
                                 preferred_element_type=jnp.float32)
        ar = sel(ar_refs[ea], ar_refs[eb])
        wr_col = jax.lax.dot_general(w32, ar, (((1,), (1,)), ((), ())),
                                     preferred_element_type=jnp.float32)
        dst32 = jnp.where(is_a,
                          nf_refs[el_idx // 2][pl.ds(i * _BN, _BN), :],
                          nf_refs[2 + el_idx // 2][pl.ds(i * _BN, _BN), :])
        er = jnp.dot(dst32, wr_col, preferred_element_type=jnp.float32)
        lg = er + el
        lg = jnp.maximum(lg, 0.2 * lg)
        adj = jnp.where(is_a, adj_refs[ea][...], adj_refs[eb][...])
        p = jnp.exp(lg) * adj
        denom = jnp.sum(p, axis=-1, keepdims=True)
        sc = jnp.where(denom > 0.0, pl.reciprocal(denom, approx=True), 1.0)
        mm = jnp.dot(p.astype(jnp.bfloat16), hs_e,
                     preferred_element_type=jnp.float32)
        bias = sel(b_refs[ea], b_refs[eb])
        h_scr[el_idx, pl.ds(i * _BN, _BN), :] = mm * sc + bias

    @pl.when(i == _G - 1)
    def _stage2():
        alpha = jax.nn.sigmoid(jnp.where(is_a, rwa[0, 0], rwb[0, 0]))
        wp, bp = sel(wpa, wpb), sel(bpa, bpb)
        wqkv = jnp.concatenate(
            [sel(wqa, wqb), sel(wka, wkb), sel(wva, wvb)], axis=1)
        wf, bf = sel(wfa, wfb) * alpha, sel(bfa, bfb) * alpha
        pe = sel(pea, peb)
        wr, br = sel(wra, wrb) * (1.0 - alpha), sel(bra, brb) * (1.0 - alpha)
        g, be = sel(ga, gb), sel(bea, beb)

        xh = []
        for t in range(_T):
            amean = []
            for r in range(_R):
                z = jnp.tanh(jnp.dot(h_scr[2 * t + r], w1_refs[t][...],
                                     preferred_element_type=jnp.float32)
                             + b1_refs[t][...])
                amean.append(jnp.mean(z, axis=0, keepdims=True))
            means = [jax.lax.dot_general(amean[r], w2_refs[t][...],
                                         (((1,), (0,)), ((), ())),
                                         preferred_element_type=jnp.float32)[0, 0]
                     for r in range(_R)]
            m = jnp.maximum(means[0], means[1])
            e0 = jnp.exp(means[0] - m)
            e1 = jnp.exp(means[1] - m)
            inv = 1.0 / (e0 + e1)
            xh.append((e0 * inv) * h_scr[2 * t] + (e1 * inv) * h_scr[2 * t + 1])

        q, k, v = [], [], []
        for t in range(_T):
            hf = jnp.dot(xh[t], wp,
                         preferred_element_type=jnp.float32) + bp + pe[t:t + 1, :]
            qkv = jnp.dot(hf, wqkv, preferred_element_type=jnp.float32)
            q.append(qkv[:, :_D])
            k.append(qkv[:, _D:2 * _D])
            v.append(qkv[:, 2 * _D:])
        for t in range(_T):
            s0 = jnp.sum(q[t] * k[0], axis=-1, keepdims=True)
            s1 = jnp.sum(q[t] * k[1], axis=-1, keepdims=True)
            m = jnp.maximum(s0, s1)
            p0 = jnp.exp(s0 - m)
            p1 = jnp.exp(s1 - m)
            inv = pl.reciprocal(p0 + p1, approx=True)
            hh = (p0 * inv) * v[0] + (p1 * inv) * v[1]
            ha = jnp.maximum(jnp.dot(hh, wf,
                                     preferred_element_type=jnp.float32) + bf, 0.0)
            res = jnp.dot(xs[t], wr, preferred_element_type=jnp.float32) + br
            y = ha + res
            mu = jnp.mean(y, axis=-1, keepdims=True)
            var = jnp.mean((y - mu) ** 2, axis=-1, keepdims=True)
            out = (y - mu) * jax.lax.rsqrt(var + 1e-5) * g + be
            o_ref[:, t * _D:(t + 1) * _D] = out


def _fused_layer(nf4, adj8, w8, al8, ar8, b8, w1s, b1s, w2s, wnt):
    full2 = lambda nt, i: (0, 0)

    def adj_spec(nt_e):
        return pl.BlockSpec(
            (_BN, _N), lambda nt, i, v=nt_e: (jnp.where(nt == v, i, 0), 0))

    return pl.pallas_call(
        _htgnn_kernel,
        out_shape=jax.ShapeDtypeStruct((_NT, _N, _T * _D), jnp.float32),
        grid=(_NT, _G),
        in_specs=(
            [pl.BlockSpec((_N, _F), full2) for _ in range(4)] +
            [adj_spec(0) for _ in range(4)] +
            [adj_spec(1) for _ in range(4)] +
            [pl.BlockSpec((_F, _D), full2) for _ in range(8)] +
            [pl.BlockSpec((1, _D), full2) for _ in range(8)] +
            [pl.BlockSpec((1, _D), full2) for _ in range(8)] +
            [pl.BlockSpec((1, _D), full2) for _ in range(8)] +
            [pl.BlockSpec((_D, _D), full2) for _ in range(2)] +
            [pl.BlockSpec((1, _D), full2) for _ in range(2)] +
            [pl.BlockSpec((_D, 1), full2) for _ in range(2)] +
            [pl.BlockSpec((_D, _D), full2) for _ in range(2)] +
            [pl.BlockSpec((1, _D), full2) for _ in range(2)] +
            [pl.BlockSpec((_D, _D), full2) for _ in range(6)] +
            [pl.BlockSpec((_D, _D), full2) for _ in range(2)] +
            [pl.BlockSpec((1, _D), full2) for _ in range(2)] +
            [pl.BlockSpec((_T, _D), full2) for _ in range(2)] +
            [pl.BlockSpec((_F, _D), full2) for _ in range(2)] +
            [pl.BlockSpec((1, _D), full2) for _ in range(2)] +
            [pl.BlockSpec((1, 1), full2) for _ in range(2)] +
            [pl.BlockSpec((1, _D), full2) for _ in range(2)] +
            [pl.BlockSpec((1, _D), full2) for _ in range(2)]
        ),
        out_specs=pl.BlockSpec((None, _N, _T * _D), lambda nt, i: (nt, 0, 0)),
        scratch_shapes=[pltpu.VMEM((_T * _R, _N, _D), jnp.float32)],
        compiler_params=pltpu.CompilerParams(
            dimension_semantics=("parallel", "arbitrary")),
    )(*nf4, *adj8, *w8, *al8, *ar8, *b8, *w1s, *b1s, *w2s, *wnt)


def kernel(nf__A__t0, nf__A__t1, nf__B__t0, nf__B__t1, adj__aa_t0, adj__ba_t0, adj__ab_t0, adj__bb_t0, adj__aa_t1, adj__ba_t1, adj__ab_t1, adj__bb_t1, intra__aa_t0__W, intra__aa_t0__attn_l, intra__aa_t0__attn_r, intra__aa_t0__bias, intra__ba_t0__W, intra__ba_t0__attn_l, intra__ba_t0__attn_r, intra__ba_t0__bias, intra__ab_t0__W, intra__ab_t0__attn_l, intra__ab_t0__attn_r, intra__ab_t0__bias, intra__bb_t0__W, intra__bb_t0__attn_l, intra__bb_t0__attn_r, intra__bb_t0__bias, intra__aa_t1__W, intra__aa_t1__attn_l, intra__aa_t1__attn_r, intra__aa_t1__bias, intra__ba_t1__W, intra__ba_t1__attn_l, intra__ba_t1__attn_r, intra__ba_t1__bias, intra__ab_t1__W, intra__ab_t1__attn_l, intra__ab_t1__attn_r, intra__ab_t1__bias, intra__bb_t1__W, intra__bb_t1__attn_l, intra__bb_t1__attn_r, intra__bb_t1__bias, inter__t0__W1, inter__t0__b1, inter__t0__w2, inter__t1__W1, inter__t1__b1, inter__t1__w2, cross__A__Wp, cross__A__bp, cross__A__Wq, cross__A__Wk, cross__A__Wv, cross__A__Wf, cross__A__bf, cross__A__pe, cross__B__Wp, cross__B__bp, cross__B__Wq, cross__B__Wk, cross__B__Wv, cross__B__Wf, cross__B__bf, cross__B__pe, res__A__W, res__A__b, res__A__res_weight, res__B__W, res__B__b, res__B__res_weight, norm__A__gamma, norm__A__beta, norm__B__gamma, norm__B__beta):
    nf4 = [nf__A__t0, nf__A__t1, nf__B__t0, nf__B__t1]
    adj8 = [adj__aa_t0, adj__ba_t0, adj__aa_t1, adj__ba_t1,
            adj__ab_t0, adj__bb_t0, adj__ab_t1, adj__bb_t1]
    w8 = [intra__aa_t0__W, intra__ba_t0__W, intra__aa_t1__W, intra__ba_t1__W,
          intra__ab_t0__W, intra__bb_t0__W, intra__ab_t1__W, intra__bb_t1__W]
    al8 = [intra__aa_t0__attn_l, intra__ba_t0__attn_l, intra__aa_t1__attn_l,
           intra__ba_t1__attn_l, intra__ab_t0__attn_l, intra__bb_t0__attn_l,
           intra__ab_t1__attn_l, intra__bb_t1__attn_l]
    ar8 = [intra__aa_t0__attn_r, intra__ba_t0__attn_r, intra__aa_t1__attn_r,
           intra__ba_t1__attn_r, intra__ab_t0__attn_r, intra__bb_t0__attn_r,
           intra__ab_t1__attn_r, intra__bb_t1__attn_r]
    b8 = [intra__aa_t0__bias, intra__ba_t0__bias, intra__aa_t1__bias,
          intra__ba_t1__bias, intra__ab_t0__bias, intra__bb_t0__bias,
          intra__ab_t1__bias, intra__bb_t1__bias]
    wnt = [cross__A__Wp, cross__B__Wp, cross__A__bp, cross__B__bp,
           cross__A__Wq, cross__B__Wq, cross__A__Wk, cross__B__Wk,
           cross__A__Wv, cross__B__Wv, cross__A__Wf, cross__B__Wf,
           cross__A__bf, cross__B__bf, cross__A__pe, cross__B__pe,
           res__A__W, res__B__W, res__A__b, res__B__b,
           res__A__res_weight, res__B__res_weight,
           norm__A__gamma, norm__B__gamma, norm__A__beta, norm__B__beta]

    out_all = _fused_layer(nf4, adj8, w8, al8, ar8, b8,
                           [inter__t0__W1, inter__t1__W1],
                           [inter__t0__b1, inter__t1__b1],
                           [inter__t0__w2, inter__t1__w2], wnt)
    return {'A': {'t0': out_all[0, :, :_D], 't1': out_all[0, :, _D:]},
            'B': {'t0': out_all[1, :, :_D], 't1': out_all[1, :, _D:]}}


# speedup vs baseline: 2.3623x; 1.0321x over previous
import jax
import jax.numpy as jnp
from jax.experimental import pallas as pl
from jax.experimental.pallas import tpu as pltpu

_N = 1024
_F = 128
_D = 128
_T = 2
_R = 2
_NT = 2
_E = 8
_BN = 256
_G = _N // _BN

_SRC_IDX = (0, 2, 1, 3, 0, 2, 1, 3)


def _htgnn_kernel(*refs):
    nf_refs = refs[0:4]
    adj_refs = refs[4:12]
    w_refs = refs[12:20]
    al_refs = refs[20:28]
    ar_refs = refs[28:36]
    b_refs = refs[36:44]
    w1_refs, b1_refs, w2_refs = refs[44:46], refs[46:48], refs[48:50]
    (wpa, wpb, bpa, bpb, wqa, wqb, wka, wkb, wva, wvb, wfa, wfb,
     bfa, bfb, pea, peb, wra, wrb, bra, brb, rwa, rwb,
     ga, gb, bea, beb) = refs[50:76]
    o_ref = refs[76]
    h_scr = refs[77]

    nt = pl.program_id(0)
    i = pl.program_id(1)

    def _stage1(rel4, dst_idx2):
        for el_idx, (adj_ref, w_ref, al_ref, ar_ref, b_ref) in enumerate(rel4):
            w32 = w_ref[...]
            src_bf = nf_refs[_SRC_IDX[el_idx]][...].astype(jnp.bfloat16)
            hs32 = jnp.dot(src_bf, w32.astype(jnp.bfloat16),
                           preferred_element_type=jnp.float32)
            hs_e = hs32.astype(jnp.bfloat16)
            el = jax.lax.dot_general(al_ref[...], hs32, (((1,), (1,)), ((), ())),
                                     preferred_element_type=jnp.float32)
            wr_col = jax.lax.dot_general(w32, ar_ref[...], (((1,), (1,)), ((), ())),
                                         preferred_element_type=jnp.float32)
            dst32 = nf_refs[dst_idx2[el_idx // 2]][pl.ds(i * _BN, _BN), :]
            er = jnp.dot(dst32, wr_col, preferred_element_type=jnp.float32)
            lg = er + el
            lg = jnp.maximum(lg, 0.2 * lg)
            p = jnp.exp(lg) * adj_ref[...]
            denom = jnp.sum(p, axis=-1, keepdims=True)
            sc = jnp.where(denom > 0.0, pl.reciprocal(denom, approx=True), 1.0)
            mm = jnp.dot(p.astype(jnp.bfloat16), hs_e,
                         preferred_element_type=jnp.float32)
            h_scr[el_idx, pl.ds(i * _BN, _BN), :] = mm * sc + b_ref[...]

    def _rel4(lo):
        return [(adj_refs[lo + k], w_refs[lo + k], al_refs[lo + k],
                 ar_refs[lo + k], b_refs[lo + k]) for k in range(4)]

    @pl.when(nt == 0)
    def _s1_a():
        _stage1(_rel4(0), (0, 1))

    @pl.when(nt == 1)
    def _s1_b():
        _stage1(_rel4(4), (2, 3))

    def _stage2(wp_r, bp_r, wq_r, wk_r, wv_r, wf_r, bf_r, pe_r,
                wr_r, br_r, rw_r, g_r, be_r, x_idx2):
        alpha = jax.nn.sigmoid(rw_r[0, 0])
        wqkv = jnp.concatenate([wq_r[...], wk_r[...], wv_r[...]], axis=1)
        wf, bf = wf_r[...] * alpha, bf_r[...] * alpha
        wr, br = wr_r[...] * (1.0 - alpha), br_r[...] * (1.0 - alpha)
        pe = pe_r[...]
        g, be = g_r[...], be_r[...]

        xh = []
        for t in range(_T):
            amean = []
            for r in range(_R):
                z = jnp.tanh(jnp.dot(h_scr[2 * t + r], w1_refs[t][...],
                                     preferred_element_type=jnp.float32)
                             + b1_refs[t][...])
                amean.append(jnp.mean(z, axis=0, keepdims=True))
            means = [jax.lax.dot_general(amean[r], w2_refs[t][...],
                                         (((1,), (0,)), ((), ())),
                                         preferred_element_type=jnp.float32)[0, 0]
                     for r in range(_R)]
            m = jnp.maximum(means[0], means[1])
            e0 = jnp.exp(means[0] - m)
            e1 = jnp.exp(means[1] - m)
            inv = 1.0 / (e0 + e1)
            xh.append((e0 * inv) * h_scr[2 * t] + (e1 * inv) * h_scr[2 * t + 1])

        q, k, v = [], [], []
        for t in range(_T):
            hf = jnp.dot(xh[t], wp_r[...],
                         preferred_element_type=jnp.float32) + bp_r[...] + pe[t:t + 1, :]
            qkv = jnp.dot(hf, wqkv, preferred_element_type=jnp.float32)
            q.append(qkv[:, :_D])
            k.append(qkv[:, _D:2 * _D])
            v.append(qkv[:, 2 * _D:])
        for t in range(_T):
            s0 = jnp.sum(q[t] * k[0], axis=-1, keepdims=True)
            s1 = jnp.sum(q[t] * k[1], axis=-1, keepdims=True)
            m = jnp.maximum(s0, s1)
            p0 = jnp.exp(s0 - m)
            p1 = jnp.exp(s1 - m)
            inv = pl.reciprocal(p0 + p1, approx=True)
            hh = (p0 * inv) * v[0] + (p1 * inv) * v[1]
            ha = jnp.maximum(jnp.dot(hh, wf,
                                     preferred_element_type=jnp.float32) + bf, 0.0)
            res = jnp.dot(nf_refs[x_idx2[t]][...], wr,
                          preferred_element_type=jnp.float32) + br
            y = ha + res
            mu = jnp.mean(y, axis=-1, keepdims=True)
            var = jnp.mean((y - mu) ** 2, axis=-1, keepdims=True)
            out = (y - mu) * jax.lax.rsqrt(var + 1e-5) * g + be
            o_ref[:, t * _D:(t + 1) * _D] = out

    last = i == _G - 1

    @pl.when(jnp.logical_and(last, nt == 0))
    def _s2_a():
        _stage2(wpa, bpa, wqa, wka, wva, wfa, bfa, pea,
                wra, bra, rwa, ga, bea, (0, 1))

    @pl.when(jnp.logical_and(last, nt == 1))
    def _s2_b():
        _stage2(wpb, bpb, wqb, wkb, wvb, wfb, bfb, peb,
                wrb, brb, rwb, gb, beb, (2, 3))


def _fused_layer(nf4, adj8, w8, al8, ar8, b8, w1s, b1s, w2s, wnt):
    full2 = lambda nt, i: (0, 0)

    def adj_spec(nt_e):
        return pl.BlockSpec(
            (_BN, _N), lambda nt, i, v=nt_e: (jnp.where(nt == v, i, 0), 0))

    return pl.pallas_call(
        _htgnn_kernel,
        out_shape=jax.ShapeDtypeStruct((_NT, _N, _T * _D), jnp.float32),
        grid=(_NT, _G),
        in_specs=(
            [pl.BlockSpec((_N, _F), full2) for _ in range(4)] +
            [adj_spec(0) for _ in range(4)] +
            [adj_spec(1) for _ in range(4)] +
            [pl.BlockSpec((_F, _D), full2) for _ in range(8)] +
            [pl.BlockSpec((1, _D), full2) for _ in range(8)] +
            [pl.BlockSpec((1, _D), full2) for _ in range(8)] +
            [pl.BlockSpec((1, _D), full2) for _ in range(8)] +
            [pl.BlockSpec((_D, _D), full2) for _ in range(2)] +
            [pl.BlockSpec((1, _D), full2) for _ in range(2)] +
            [pl.BlockSpec((_D, 1), full2) for _ in range(2)] +
            [pl.BlockSpec((_D, _D), full2) for _ in range(2)] +
            [pl.BlockSpec((1, _D), full2) for _ in range(2)] +
            [pl.BlockSpec((_D, _D), full2) for _ in range(6)] +
            [pl.BlockSpec((_D, _D), full2) for _ in range(2)] +
            [pl.BlockSpec((1, _D), full2) for _ in range(2)] +
            [pl.BlockSpec((_T, _D), full2) for _ in range(2)] +
            [pl.BlockSpec((_F, _D), full2) for _ in range(2)] +
            [pl.BlockSpec((1, _D), full2) for _ in range(2)] +
            [pl.BlockSpec((1, 1), full2) for _ in range(2)] +
            [pl.BlockSpec((1, _D), full2) for _ in range(2)] +
            [pl.BlockSpec((1, _D), full2) for _ in range(2)]
        ),
        out_specs=pl.BlockSpec((None, _N, _T * _D), lambda nt, i: (nt, 0, 0)),
        scratch_shapes=[pltpu.VMEM((_T * _R, _N, _D), jnp.float32)],
        compiler_params=pltpu.CompilerParams(
            dimension_semantics=("parallel", "arbitrary")),
    )(*nf4, *adj8, *w8, *al8, *ar8, *b8, *w1s, *b1s, *w2s, *wnt)


def kernel(nf__A__t0, nf__A__t1, nf__B__t0, nf__B__t1, adj__aa_t0, adj__ba_t0, adj__ab_t0, adj__bb_t0, adj__aa_t1, adj__ba_t1, adj__ab_t1, adj__bb_t1, intra__aa_t0__W, intra__aa_t0__attn_l, intra__aa_t0__attn_r, intra__aa_t0__bias, intra__ba_t0__W, intra__ba_t0__attn_l, intra__ba_t0__attn_r, intra__ba_t0__bias, intra__ab_t0__W, intra__ab_t0__attn_l, intra__ab_t0__attn_r, intra__ab_t0__bias, intra__bb_t0__W, intra__bb_t0__attn_l, intra__bb_t0__attn_r, intra__bb_t0__bias, intra__aa_t1__W, intra__aa_t1__attn_l, intra__aa_t1__attn_r, intra__aa_t1__bias, intra__ba_t1__W, intra__ba_t1__attn_l, intra__ba_t1__attn_r, intra__ba_t1__bias, intra__ab_t1__W, intra__ab_t1__attn_l, intra__ab_t1__attn_r, intra__ab_t1__bias, intra__bb_t1__W, intra__bb_t1__attn_l, intra__bb_t1__attn_r, intra__bb_t1__bias, inter__t0__W1, inter__t0__b1, inter__t0__w2, inter__t1__W1, inter__t1__b1, inter__t1__w2, cross__A__Wp, cross__A__bp, cross__A__Wq, cross__A__Wk, cross__A__Wv, cross__A__Wf, cross__A__bf, cross__A__pe, cross__B__Wp, cross__B__bp, cross__B__Wq, cross__B__Wk, cross__B__Wv, cross__B__Wf, cross__B__bf, cross__B__pe, res__A__W, res__A__b, res__A__res_weight, res__B__W, res__B__b, res__B__res_weight, norm__A__gamma, norm__A__beta, norm__B__gamma, norm__B__beta):
    nf4 = [nf__A__t0, nf__A__t1, nf__B__t0, nf__B__t1]
    adj8 = [adj__aa_t0, adj__ba_t0, adj__aa_t1, adj__ba_t1,
            adj__ab_t0, adj__bb_t0, adj__ab_t1, adj__bb_t1]
    w8 = [intra__aa_t0__W, intra__ba_t0__W, intra__aa_t1__W, intra__ba_t1__W,
          intra__ab_t0__W, intra__bb_t0__W, intra__ab_t1__W, intra__bb_t1__W]
    al8 = [intra__aa_t0__attn_l, intra__ba_t0__attn_l, intra__aa_t1__attn_l,
           intra__ba_t1__attn_l, intra__ab_t0__attn_l, intra__bb_t0__attn_l,
           intra__ab_t1__attn_l, intra__bb_t1__attn_l]
    ar8 = [intra__aa_t0__attn_r, intra__ba_t0__attn_r, intra__aa_t1__attn_r,
           intra__ba_t1__attn_r, intra__ab_t0__attn_r, intra__bb_t0__attn_r,
           intra__ab_t1__attn_r, intra__bb_t1__attn_r]
    b8 = [intra__aa_t0__bias, intra__ba_t0__bias, intra__aa_t1__bias,
          intra__ba_t1__bias, intra__ab_t0__bias, intra__bb_t0__bias,
          intra__ab_t1__bias, intra__bb_t1__bias]
    wnt = [cross__A__Wp, cross__B__Wp, cross__A__bp, cross__B__bp,
           cross__A__Wq, cross__B__Wq, cross__A__Wk, cross__B__Wk,
           cross__A__Wv, cross__B__Wv, cross__A__Wf, cross__B__Wf,
           cross__A__bf, cross__B__bf, cross__A__pe, cross__B__pe,
           res__A__W, res__B__W, res__A__b, res__B__b,
           res__A__res_weight, res__B__res_weight,
           norm__A__gamma, norm__B__gamma, norm__A__beta, norm__B__beta]

    out_all = _fused_layer(nf4, adj8, w8, al8, ar8, b8,
                           [inter__t0__W1, inter__t1__W1],
                           [inter__t0__b1, inter__t1__b1],
                           [inter__t0__w2, inter__t1__w2], wnt)
    return {'A': {'t0': out_all[0, :, :_D], 't1': out_all[0, :, _D:]},
            'B': {'t0': out_all[1, :, :_D], 't1': out_all[1, :, _D:]}}


# speedup vs baseline: 2.7796x; 1.1766x over previous
import jax
import jax.numpy as jnp
from jax.experimental import pallas as pl
from jax.experimental.pallas import tpu as pltpu

_N = 1024
_F = 128
_D = 128
_T = 2
_R = 2
_NT = 2
_E = 8
_BN = 512
_G = _N // _BN

_SRC_IDX = (0, 2, 1, 3, 0, 2, 1, 3)


def _htgnn_kernel(*refs):
    nf_refs = refs[0:4]
    adj_refs = refs[4:12]
    w_refs = refs[12:20]
    al_refs = refs[20:28]
    ar_refs = refs[28:36]
    b_refs = refs[36:44]
    w1_refs, b1_refs, w2_refs = refs[44:46], refs[46:48], refs[48:50]
    (wpa, wpb, bpa, bpb, wqa, wqb, wka, wkb, wva, wvb, wfa, wfb,
     bfa, bfb, pea, peb, wra, wrb, bra, brb, rwa, rwb,
     ga, gb, bea, beb) = refs[50:76]
    o_ref = refs[76]
    h_scr = refs[77]

    nt = pl.program_id(0)
    i = pl.program_id(1)

    def _stage1(rel4, dst_idx2):
        for el_idx, (adj_ref, w_ref, al_ref, ar_ref, b_ref) in enumerate(rel4):
            w32 = w_ref[...]
            src_bf = nf_refs[_SRC_IDX[el_idx]][...].astype(jnp.bfloat16)
            hs32 = jnp.dot(src_bf, w32.astype(jnp.bfloat16),
                           preferred_element_type=jnp.float32)
            hs_e = hs32.astype(jnp.bfloat16)
            el = jax.lax.dot_general(al_ref[...], hs32, (((1,), (1,)), ((), ())),
                                     preferred_element_type=jnp.float32)
            wr_col = jax.lax.dot_general(w32, ar_ref[...], (((1,), (1,)), ((), ())),
                                         preferred_element_type=jnp.float32)
            dst32 = nf_refs[dst_idx2[el_idx // 2]][pl.ds(i * _BN, _BN), :]
            er = jnp.dot(dst32, wr_col, preferred_element_type=jnp.float32)
            lg = er + el
            lg = jnp.maximum(lg, 0.2 * lg)
            p = jnp.exp(lg) * adj_ref[...]
            denom = jnp.sum(p, axis=-1, keepdims=True)
            sc = jnp.where(denom > 0.0, pl.reciprocal(denom, approx=True), 1.0)
            mm = jnp.dot(p.astype(jnp.bfloat16), hs_e,
                         preferred_element_type=jnp.float32)
            h_scr[el_idx, pl.ds(i * _BN, _BN), :] = mm * sc + b_ref[...]

    def _rel4(lo):
        return [(adj_refs[lo + k], w_refs[lo + k], al_refs[lo + k],
                 ar_refs[lo + k], b_refs[lo + k]) for k in range(4)]

    @pl.when(nt == 0)
    def _s1_a():
        _stage1(_rel4(0), (0, 1))

    @pl.when(nt == 1)
    def _s1_b():
        _stage1(_rel4(4), (2, 3))

    def _stage2(wp_r, bp_r, wq_r, wk_r, wv_r, wf_r, bf_r, pe_r,
                wr_r, br_r, rw_r, g_r, be_r, x_idx2):
        alpha = jax.nn.sigmoid(rw_r[0, 0])
        wqkv = jnp.concatenate([wq_r[...], wk_r[...], wv_r[...]], axis=1)
        wf, bf = wf_r[...] * alpha, bf_r[...] * alpha
        wr, br = wr_r[...] * (1.0 - alpha), br_r[...] * (1.0 - alpha)
        pe = pe_r[...]
        g, be = g_r[...], be_r[...]

        xh = []
        for t in range(_T):
            amean = []
            for r in range(_R):
                z = jnp.tanh(jnp.dot(h_scr[2 * t + r], w1_refs[t][...],
                                     preferred_element_type=jnp.float32)
                             + b1_refs[t][...])
                amean.append(jnp.mean(z, axis=0, keepdims=True))
            means = [jax.lax.dot_general(amean[r], w2_refs[t][...],
                                         (((1,), (0,)), ((), ())),
                                         preferred_element_type=jnp.float32)[0, 0]
                     for r in range(_R)]
            m = jnp.maximum(means[0], means[1])
            e0 = jnp.exp(means[0] - m)
            e1 = jnp.exp(means[1] - m)
            inv = 1.0 / (e0 + e1)
            xh.append((e0 * inv) * h_scr[2 * t] + (e1 * inv) * h_scr[2 * t + 1])

        q, k, v = [], [], []
        for t in range(_T):
            hf = jnp.dot(xh[t], wp_r[...],
                         preferred_element_type=jnp.float32) + bp_r[...] + pe[t:t + 1, :]
            qkv = jnp.dot(hf, wqkv, preferred_element_type=jnp.float32)
            q.append(qkv[:, :_D])
            k.append(qkv[:, _D:2 * _D])
            v.append(qkv[:, 2 * _D:])
        for t in range(_T):
            s0 = jnp.sum(q[t] * k[0], axis=-1, keepdims=True)
            s1 = jnp.sum(q[t] * k[1], axis=-1, keepdims=True)
            m = jnp.maximum(s0, s1)
            p0 = jnp.exp(s0 - m)
            p1 = jnp.exp(s1 - m)
            inv = pl.reciprocal(p0 + p1, approx=True)
            hh = (p0 * inv) * v[0] + (p1 * inv) * v[1]
            ha = jnp.maximum(jnp.dot(hh, wf,
                                     preferred_element_type=jnp.float32) + bf, 0.0)
            res = jnp.dot(nf_refs[x_idx2[t]][...], wr,
                          preferred_element_type=jnp.float32) + br
            y = ha + res
            mu = jnp.mean(y, axis=-1, keepdims=True)
            var = jnp.mean((y - mu) ** 2, axis=-1, keepdims=True)
            out = (y - mu) * jax.lax.rsqrt(var + 1e-5) * g + be
            o_ref[:, t * _D:(t + 1) * _D] = out

    last = i == _G - 1

    @pl.when(jnp.logical_and(last, nt == 0))
    def _s2_a():
        _stage2(wpa, bpa, wqa, wka, wva, wfa, bfa, pea,
                wra, bra, rwa, ga, bea, (0, 1))

    @pl.when(jnp.logical_and(last, nt == 1))
    def _s2_b():
        _stage2(wpb, bpb, wqb, wkb, wvb, wfb, bfb, peb,
                wrb, brb, rwb, gb, beb, (2, 3))


def _fused_layer(nf4, adj8, w8, al8, ar8, b8, w1s, b1s, w2s, wnt):
    full2 = lambda nt, i: (0, 0)

    def adj_spec(nt_e):
        return pl.BlockSpec(
            (_BN, _N), lambda nt, i, v=nt_e: (jnp.where(nt == v, i, 0), 0))

    return pl.pallas_call(
        _htgnn_kernel,
        out_shape=jax.ShapeDtypeStruct((_NT, _N, _T * _D), jnp.float32),
        grid=(_NT, _G),
        in_specs=(
            [pl.BlockSpec((_N, _F), full2) for _ in range(4)] +
            [adj_spec(0) for _ in range(4)] +
            [adj_spec(1) for _ in range(4)] +
            [pl.BlockSpec((_F, _D), full2) for _ in range(8)] +
            [pl.BlockSpec((1, _D), full2) for _ in range(8)] +
            [pl.BlockSpec((1, _D), full2) for _ in range(8)] +
            [pl.BlockSpec((1, _D), full2) for _ in range(8)] +
            [pl.BlockSpec((_D, _D), full2) for _ in range(2)] +
            [pl.BlockSpec((1, _D), full2) for _ in range(2)] +
            [pl.BlockSpec((_D, 1), full2) for _ in range(2)] +
            [pl.BlockSpec((_D, _D), full2) for _ in range(2)] +
            [pl.BlockSpec((1, _D), full2) for _ in range(2)] +
            [pl.BlockSpec((_D, _D), full2) for _ in range(6)] +
            [pl.BlockSpec((_D, _D), full2) for _ in range(2)] +
            [pl.BlockSpec((1, _D), full2) for _ in range(2)] +
            [pl.BlockSpec((_T, _D), full2) for _ in range(2)] +
            [pl.BlockSpec((_F, _D), full2) for _ in range(2)] +
            [pl.BlockSpec((1, _D), full2) for _ in range(2)] +
            [pl.BlockSpec((1, 1), full2) for _ in range(2)] +
            [pl.BlockSpec((1, _D), full2) for _ in range(2)] +
            [pl.BlockSpec((1, _D), full2) for _ in range(2)]
        ),
        out_specs=pl.BlockSpec((None, _N, _T * _D), lambda nt, i: (nt, 0, 0)),
        scratch_shapes=[pltpu.VMEM((_T * _R, _N, _D), jnp.float32)],
        compiler_params=pltpu.CompilerParams(
            dimension_semantics=("parallel", "arbitrary")),
    )(*nf4, *adj8, *w8, *al8, *ar8, *b8, *w1s, *b1s, *w2s, *wnt)


def kernel(nf__A__t0, nf__A__t1, nf__B__t0, nf__B__t1, adj__aa_t0, adj__ba_t0, adj__ab_t0, adj__bb_t0, adj__aa_t1, adj__ba_t1, adj__ab_t1, adj__bb_t1, intra__aa_t0__W, intra__aa_t0__attn_l, intra__aa_t0__attn_r, intra__aa_t0__bias, intra__ba_t0__W, intra__ba_t0__attn_l, intra__ba_t0__attn_r, intra__ba_t0__bias, intra__ab_t0__W, intra__ab_t0__attn_l, intra__ab_t0__attn_r, intra__ab_t0__bias, intra__bb_t0__W, intra__bb_t0__attn_l, intra__bb_t0__attn_r, intra__bb_t0__bias, intra__aa_t1__W, intra__aa_t1__attn_l, intra__aa_t1__attn_r, intra__aa_t1__bias, intra__ba_t1__W, intra__ba_t1__attn_l, intra__ba_t1__attn_r, intra__ba_t1__bias, intra__ab_t1__W, intra__ab_t1__attn_l, intra__ab_t1__attn_r, intra__ab_t1__bias, intra__bb_t1__W, intra__bb_t1__attn_l, intra__bb_t1__attn_r, intra__bb_t1__bias, inter__t0__W1, inter__t0__b1, inter__t0__w2, inter__t1__W1, inter__t1__b1, inter__t1__w2, cross__A__Wp, cross__A__bp, cross__A__Wq, cross__A__Wk, cross__A__Wv, cross__A__Wf, cross__A__bf, cross__A__pe, cross__B__Wp, cross__B__bp, cross__B__Wq, cross__B__Wk, cross__B__Wv, cross__B__Wf, cross__B__bf, cross__B__pe, res__A__W, res__A__b, res__A__res_weight, res__B__W, res__B__b, res__B__res_weight, norm__A__gamma, norm__A__beta, norm__B__gamma, norm__B__beta):
    nf4 = [nf__A__t0, nf__A__t1, nf__B__t0, nf__B__t1]
    adj8 = [adj__aa_t0, adj__ba_t0, adj__aa_t1, adj__ba_t1,
            adj__ab_t0, adj__bb_t0, adj__ab_t1, adj__bb_t1]
    w8 = [intra__aa_t0__W, intra__ba_t0__W, intra__aa_t1__W, intra__ba_t1__W,
          intra__ab_t0__W, intra__bb_t0__W, intra__ab_t1__W, intra__bb_t1__W]
    al8 = [intra__aa_t0__attn_l, intra__ba_t0__attn_l, intra__aa_t1__attn_l,
           intra__ba_t1__attn_l, intra__ab_t0__attn_l, intra__bb_t0__attn_l,
           intra__ab_t1__attn_l, intra__bb_t1__attn_l]
    ar8 = [intra__aa_t0__attn_r, intra__ba_t0__attn_r, intra__aa_t1__attn_r,
           intra__ba_t1__attn_r, intra__ab_t0__attn_r, intra__bb_t0__attn_r,
           intra__ab_t1__attn_r, intra__bb_t1__attn_r]
    b8 = [intra__aa_t0__bias, intra__ba_t0__bias, intra__aa_t1__bias,
          intra__ba_t1__bias, intra__ab_t0__bias, intra__bb_t0__bias,
          intra__ab_t1__bias, intra__bb_t1__bias]
    wnt = [cross__A__Wp, cross__B__Wp, cross__A__bp, cross__B__bp,
           cross__A__Wq, cross__B__Wq, cross__A__Wk, cross__B__Wk,
           cross__A__Wv, cross__B__Wv, cross__A__Wf, cross__B__Wf,
           cross__A__bf, cross__B__bf, cross__A__pe, cross__B__pe,
           res__A__W, res__B__W, res__A__b, res__B__b,
           res__A__res_weight, res__B__res_weight,
           norm__A__gamma, norm__B__gamma, norm__A__beta, norm__B__beta]

    out_all = _fused_layer(nf4, adj8, w8, al8, ar8, b8,
                           [inter__t0__W1, inter__t1__W1],
                           [inter__t0__b1, inter__t1__b1],
                           [inter__t0__w2, inter__t1__w2], wnt)
    return {'A': {'t0': out_all[0, :, :_D], 't1': out_all[0, :, _D:]},
            'B': {'t0': out_all[1, :, :_D], 't1': out_all[1, :, _D:]}}
